# baseline (speedup 1.0000x reference)
import functools

import jax
import jax.numpy as jnp
from jax import lax
from jax.experimental import pallas as pl
from jax.experimental.pallas import tpu as pltpu
from jax.experimental.pallas import tpu_sc as plsc


@functools.cache
def _build(B, F, V, D):
    info = plsc.get_sparse_core_info()
    NC, NS, L = info.num_cores, info.num_subcores, info.num_lanes
    NW = NC * NS
    assert D == NW, "one emb_dim plane per vector subcore"
    CS = 6400
    NK = 16
    VA = (V // 128) * 128
    TAIL = V - VA
    assert TAIL % L == 0 and (NK - 1) * CS >= VA - CS
    CHO = 2048
    NCH = B // CHO
    mesh = plsc.VectorSubcoreMesh(core_axis_name="c", subcore_axis_name="s")

    def chunk_off(k):
        return min(k * CS, VA - CS)

    @functools.partial(
        pl.kernel,
        mesh=mesh,
        out_type=jax.ShapeDtypeStruct((F, D, B), jnp.float32),
        scratch_types=[
            pltpu.VMEM_SHARED((NS, CS), jnp.float32),
            pltpu.VMEM_SHARED((NS, CS), jnp.float32),
            pltpu.VMEM((V,), jnp.float32),
            pltpu.VMEM((TAIL,), jnp.float32),
            pltpu.VMEM((2, CHO), jnp.int32),
            pltpu.VMEM((2, CHO), jnp.float32),
            pltpu.VMEM((D,), jnp.float32),
            pltpu.SemaphoreType.DMA,
            pltpu.SemaphoreType.DMA,
            pltpu.SemaphoreType.DMA,
            pltpu.SemaphoreType.DMA,
            pltpu.SemaphoreType.DMA,
            pltpu.SemaphoreType.DMA,
        ],
        compiler_params=pltpu.CompilerParams(needs_layout_passes=False),
    )
    def gather_bias(tab_t, x_t, col, tail3, out_t, spm_a, spm_b, plane_v,
                    tail_v, idx_v, out_v, col_v, sem_ca, sem_cb, sem_t,
                    sem_i0, sem_i1, sem_o):
        c = lax.axis_index("c")
        s = lax.axis_index("s")
        w = s * NC + c
        w16 = jnp.full((L,), w, jnp.int32)
        sem_i = (sem_i0, sem_i1)
        slots = ((spm_a, sem_ca), (spm_b, sem_cb))

        def chunk_cp(f, k):
            slot, sem = slots[k % 2]
            return pltpu.make_async_copy(
                tab_t.at[f, w].at[pl.ds(chunk_off(k), CS)],
                slot.at[s], sem)

        def field_body(f, carry):
            chunk_cp(f, 0).start()
            chunk_cp(f, 1).start()
            pltpu.async_copy(tail3.at[f, w], tail_v, sem_t).wait()
            for t in range(TAIL // L):
                plane_v[pl.ds(VA + t * L, L)] = tail_v[pl.ds(t * L, L)]
            for k in range(NK):
                chunk_cp(f, k).wait()
                slot, _ = slots[k % 2]
                pltpu.sync_copy(slot.at[s],
                                plane_v.at[pl.ds(chunk_off(k), CS)])
                if k + 2 < NK:
                    chunk_cp(f, k + 2).start()

            pltpu.sync_copy(col.at[f], col_v)
            bias = plsc.load_gather(col_v, [w16])
            idx_cp = [None, None]
            out_cp = [None, None]
            idx_cp[0] = pltpu.async_copy(
                x_t.at[f, pl.ds(0, CHO)], idx_v.at[0], sem_i[0])
            for cch in range(NCH):
                b0 = cch % 2
                idx_cp[b0].wait()
                if cch + 1 < NCH:
                    b1 = (cch + 1) % 2
                    idx_cp[b1] = pltpu.async_copy(
                        x_t.at[f, pl.ds((cch + 1) * CHO, CHO)],
                        idx_v.at[b1], sem_i[b1])
                if out_cp[b0] is not None:
                    out_cp[b0].wait()

                @plsc.parallel_loop(0, CHO // L, unroll=8)
                def ibody(i):
                    idx16 = idx_v[b0, pl.ds(i * L, L)]
                    out_v[b0, pl.ds(i * L, L)] = (
                        plsc.load_gather(plane_v, [idx16]) + bias
                    )

                out_cp[b0] = pltpu.async_copy(
                    out_v.at[b0], out_t.at[f, w, pl.ds(cch * CHO, CHO)],
                    sem_o)
            out_cp[0].wait()
            out_cp[1].wait()
            return carry

        lax.fori_loop(0, F, field_body, 0)

    return gather_bias


def kernel(x_cat, tables, col_embed):
    F, V, D = tables.shape
    B = x_cat.shape[0]
    tab_t = tables.transpose(0, 2, 1)
    x_t = x_cat.astype(jnp.int32).T
    VA = (V // 128) * 128
    tail3 = tab_t[:, :, VA:]
    out_t = _build(B, F, V, D)(tab_t, x_t, col_embed, tail3)
    return out_t.transpose(2, 0, 1)


# speedup vs baseline: 1.4389x; 1.4389x over previous
import functools

import jax
import jax.numpy as jnp
from jax import lax
from jax.experimental import pallas as pl
from jax.experimental.pallas import tpu as pltpu
from jax.experimental.pallas import tpu_sc as plsc


@functools.cache
def _build(B, F, V, D):
    info = plsc.get_sparse_core_info()
    NC, NS, L = info.num_cores, info.num_subcores, info.num_lanes
    NW = NC * NS
    assert D == NW, "one emb_dim plane per vector subcore"
    CHO = 2048
    NCH = B // CHO
    assert B % CHO == 0 and CHO % L == 0
    mesh = plsc.VectorSubcoreMesh(core_axis_name="c", subcore_axis_name="s")

    @functools.partial(
        pl.kernel,
        mesh=mesh,
        out_type=jax.ShapeDtypeStruct((F, D, B), jnp.float32),
        scratch_types=[
            pltpu.VMEM((V,), jnp.float32),
            pltpu.VMEM((2, CHO), jnp.int32),
            pltpu.VMEM((2, CHO), jnp.float32),
            pltpu.VMEM((D,), jnp.float32),
            pltpu.SemaphoreType.DMA,
            pltpu.SemaphoreType.DMA,
            pltpu.SemaphoreType.DMA,
            pltpu.SemaphoreType.DMA,
        ],
        compiler_params=pltpu.CompilerParams(needs_layout_passes=False),
    )
    def gather_bias(tab_t, x_t, col, out_t, plane_v, idx_v, out_v, col_v,
                    sem_i0, sem_i1, sem_o0, sem_o1):
        w = lax.axis_index("s") * NC + lax.axis_index("c")
        w16 = jnp.full((L,), w, jnp.int32)
        sem_i = (sem_i0, sem_i1)
        sem_o = (sem_o0, sem_o1)

        def field_body(f, carry):
            pltpu.sync_copy(tab_t.at[f, w], plane_v)
            pltpu.sync_copy(col.at[f], col_v)
            bias = plsc.load_gather(col_v, [w16])
            idx_cp = [None, None]
            out_cp = [None, None]
            idx_cp[0] = pltpu.async_copy(
                x_t.at[f, pl.ds(0, CHO)], idx_v.at[0], sem_i[0])
            for c in range(NCH):
                b0 = c % 2
                idx_cp[b0].wait()
                if c + 1 < NCH:
                    b1 = (c + 1) % 2
                    idx_cp[b1] = pltpu.async_copy(
                        x_t.at[f, pl.ds((c + 1) * CHO, CHO)],
                        idx_v.at[b1], sem_i[b1])
                if out_cp[b0] is not None:
                    out_cp[b0].wait()

                @plsc.parallel_loop(0, CHO // L, unroll=8)
                def ibody(i):
                    idx16 = idx_v[b0, pl.ds(i * L, L)]
                    out_v[b0, pl.ds(i * L, L)] = (
                        plsc.load_gather(plane_v, [idx16]) + bias
                    )

                out_cp[b0] = pltpu.async_copy(
                    out_v.at[b0], out_t.at[f, w, pl.ds(c * CHO, CHO)],
                    sem_o[b0])
            out_cp[0].wait()
            out_cp[1].wait()
            return carry

        lax.fori_loop(0, F, field_body, 0)

    return gather_bias


def kernel(x_cat, tables, col_embed):
    F, V, D = tables.shape
    B = x_cat.shape[0]
    tab_t = tables.transpose(0, 2, 1)
    x_t = x_cat.astype(jnp.int32).T
    out_t = _build(B, F, V, D)(tab_t, x_t, col_embed)
    return out_t.transpose(2, 0, 1)
